# baseline (speedup 1.0000x reference)
import functools

import jax
import jax.numpy as jnp
from jax import lax
from jax.experimental import pallas as pl
from jax.experimental.pallas import tpu as pltpu
from jax.experimental.pallas import tpu_sc as plsc

_N = 4194304
_NW = 32
_SC_TOT = 2621440
_PER_W = _SC_TOT // _NW
_CHUNK = 8192
_NCHUNK = _PER_W // _CHUNK
_L = 16
_UNROLL = 8
_NBUF = 4


def _sc_body(inp_hbm, tgt_hbm, out_hbm,
             ib0, ib1, ib2, ib3, tb0, tb1, tb2, tb3, accv, s0, s1, s2, s3):
    wid = lax.axis_index("s") * 2 + lax.axis_index("c")
    base = pl.multiple_of(wid * _PER_W, _PER_W)
    ibufs = (ib0, ib1, ib2, ib3)
    tbufs = (tb0, tb1, tb2, tb3)
    sems = (s0, s1, s2, s3)
    h_i = [None] * _NBUF
    h_t = [None] * _NBUF
    for c in range(_NBUF - 1):
        off = base + c * _CHUNK
        h_i[c] = pltpu.async_copy(
            inp_hbm.at[pl.ds(off, _CHUNK)], ibufs[c], sems[c])
        h_t[c] = pltpu.async_copy(
            tgt_hbm.at[pl.ds(off, _CHUNK)], tbufs[c], sems[c])
    acc = jnp.zeros((_L,), jnp.float32)
    for c in range(_NCHUNK):
        cur = c % _NBUF
        nxt = (c + _NBUF - 1) % _NBUF
        if c + _NBUF - 1 < _NCHUNK:
            off = base + (c + _NBUF - 1) * _CHUNK
            h_i[nxt] = pltpu.async_copy(
                inp_hbm.at[pl.ds(off, _CHUNK)], ibufs[nxt], sems[nxt])
            h_t[nxt] = pltpu.async_copy(
                tgt_hbm.at[pl.ds(off, _CHUNK)], tbufs[nxt], sems[nxt])
        h_i[cur].wait()
        h_t[cur].wait()
        ibuf = ibufs[cur]
        tbuf = tbufs[cur]

        def _vec_body(i, a, ibuf=ibuf, tbuf=tbuf):
            j = i * (_UNROLL * _L)
            for u in range(_UNROLL):
                x = ibuf[pl.ds(j + u * _L, _L)]
                t = tbuf[pl.ds(j + u * _L, _L)]
                d = t - x
                a = a + d * d
            return a

        acc = lax.fori_loop(0, _CHUNK // (_UNROLL * _L), _vec_body, acc)
    accv[...] = acc * (1.0 / _N)
    pltpu.sync_copy(accv, out_hbm.at[wid])


_sc_mse = functools.partial(
    pl.kernel,
    mesh=plsc.VectorSubcoreMesh(core_axis_name="c", subcore_axis_name="s"),
    out_type=jax.ShapeDtypeStruct((_NW, _L), jnp.float32),
    scratch_types=(
        [pltpu.VMEM((_CHUNK,), jnp.float32)] * 8
        + [pltpu.VMEM((_L,), jnp.float32)]
        + [pltpu.SemaphoreType.DMA] * 4
    ),
)(_sc_body)


_TC_COLS = 1024
_ALL_ROWS = _N // _TC_COLS
_SC_ELEMS = 2621440
_SC_ROWS = _SC_ELEMS // _TC_COLS
_TC_BLOCK = 256
_TC_OFF = _SC_ROWS // _TC_BLOCK
_TC_GRID = (_ALL_ROWS - _SC_ROWS) // _TC_BLOCK


def _tc_body(i_ref, t_ref, o_ref):
    @pl.when(pl.program_id(0) == 0)
    def _init():
        o_ref[...] = jnp.zeros_like(o_ref)

    d = t_ref[...] - i_ref[...]
    o_ref[...] += (jnp.sum(d * d) * (1.0 / _N)).reshape(1, 1)


def kernel(input, target):
    tc_part = pl.pallas_call(
        _tc_body,
        grid=(_TC_GRID,),
        in_specs=[
            pl.BlockSpec((_TC_BLOCK, _TC_COLS), lambda i: (i + _TC_OFF, 0)),
            pl.BlockSpec((_TC_BLOCK, _TC_COLS), lambda i: (i + _TC_OFF, 0)),
        ],
        out_specs=pl.BlockSpec((1, 1), lambda i: (0, 0)),
        out_shape=jax.ShapeDtypeStruct((1, 1), jnp.float32),
    )(input.reshape(_ALL_ROWS, _TC_COLS),
      target.reshape(_ALL_ROWS, _TC_COLS))
    parts = _sc_mse(input, target)
    return jnp.sum(parts) + tc_part[0, 0]


# speedup vs baseline: 1.8172x; 1.8172x over previous
import functools

import jax
import jax.numpy as jnp
from jax import lax
from jax.experimental import pallas as pl
from jax.experimental.pallas import tpu as pltpu
from jax.experimental.pallas import tpu_sc as plsc

_N = 4194304
_NW = 32
_PER_W = _N // _NW
_CHUNK = 8192
_NCHUNK = _PER_W // _CHUNK
_L = 16
_UNROLL = 8
_NBUF = 6


def _sc_body(inp_hbm, tgt_hbm, out_hbm,
             ib0, ib1, ib2, ib3, ib4, ib5, tb0, tb1, tb2, tb3, tb4, tb5,
             accv, s0, s1, s2, s3, s4, s5):
    wid = lax.axis_index("s") * 2 + lax.axis_index("c")
    base = pl.multiple_of(wid * _PER_W, _PER_W)
    ibufs = (ib0, ib1, ib2, ib3, ib4, ib5)
    tbufs = (tb0, tb1, tb2, tb3, tb4, tb5)
    sems = (s0, s1, s2, s3, s4, s5)
    h_i = [None] * _NBUF
    h_t = [None] * _NBUF
    for c in range(_NBUF - 1):
        off = base + c * _CHUNK
        h_i[c] = pltpu.async_copy(
            inp_hbm.at[pl.ds(off, _CHUNK)], ibufs[c], sems[c])
        h_t[c] = pltpu.async_copy(
            tgt_hbm.at[pl.ds(off, _CHUNK)], tbufs[c], sems[c])
    acc = jnp.zeros((_L,), jnp.float32)
    for c in range(_NCHUNK):
        cur = c % _NBUF
        nxt = (c + _NBUF - 1) % _NBUF
        if c + _NBUF - 1 < _NCHUNK:
            off = base + (c + _NBUF - 1) * _CHUNK
            h_i[nxt] = pltpu.async_copy(
                inp_hbm.at[pl.ds(off, _CHUNK)], ibufs[nxt], sems[nxt])
            h_t[nxt] = pltpu.async_copy(
                tgt_hbm.at[pl.ds(off, _CHUNK)], tbufs[nxt], sems[nxt])
        h_i[cur].wait()
        h_t[cur].wait()
        ibuf = ibufs[cur]
        tbuf = tbufs[cur]

        def _vec_body(i, a, ibuf=ibuf, tbuf=tbuf):
            j = i * (_UNROLL * _L)
            for u in range(_UNROLL):
                x = ibuf[pl.ds(j + u * _L, _L)]
                t = tbuf[pl.ds(j + u * _L, _L)]
                d = t - x
                a = a + d * d
            return a

        acc = lax.fori_loop(0, _CHUNK // (_UNROLL * _L), _vec_body, acc)
    accv[...] = acc * (1.0 / _N)
    pltpu.sync_copy(accv, out_hbm.at[wid])


_sc_mse = functools.partial(
    pl.kernel,
    mesh=plsc.VectorSubcoreMesh(core_axis_name="c", subcore_axis_name="s"),
    out_type=jax.ShapeDtypeStruct((_NW, _L), jnp.float32),
    scratch_types=(
        [pltpu.VMEM((_CHUNK,), jnp.float32)] * 12
        + [pltpu.VMEM((_L,), jnp.float32)]
        + [pltpu.SemaphoreType.DMA] * 6
    ),
)(_sc_body)


def _final_body(p_ref, o_ref):
    o_ref[...] = jnp.sum(p_ref[...]).reshape(1, 1)


def kernel(input, target):
    parts = _sc_mse(input, target)
    out = pl.pallas_call(
        _final_body,
        out_shape=jax.ShapeDtypeStruct((1, 1), jnp.float32),
    )(parts)
    return out[0, 0]
